# Initial kernel scaffold; baseline (speedup 1.0000x reference)
import jax
import jax.numpy as jnp
from jax.experimental import pallas as pl


def kernel(x, W):
    raise NotImplementedError("write your pallas kernel here")


# speedup vs baseline: 4.1339x; 4.1339x over previous
import functools

import jax
import jax.numpy as jnp
from jax import lax
from jax.experimental import pallas as pl
from jax.experimental.pallas import tpu as pltpu
from jax.experimental.pallas import tpu_sc as plsc

_ROWS = 16384
_COLS = 26
_DIM = 3
_TABLE_ROWS = 10
_N = _ROWS * _COLS
_NW = 32
_CHUNK = _N // _NW
_NVEC = _CHUNK // 16


@functools.partial(
    pl.kernel,
    out_type=jax.ShapeDtypeStruct((_N * _DIM,), jnp.float32),
    mesh=plsc.VectorSubcoreMesh(core_axis_name="c", subcore_axis_name="s"),
    compiler_params=pltpu.CompilerParams(needs_layout_passes=False),
    scratch_types=[
        pltpu.VMEM((_CHUNK,), jnp.int32),
        pltpu.VMEM((_CHUNK * _DIM,), jnp.float32),
        pltpu.VMEM((32,), jnp.float32),
    ],
)
def _sc_lookup(x_hbm, w_hbm, out_hbm, idx_v, out_v, w_v):
    wid = lax.axis_index("s") * 2 + lax.axis_index("c")
    base = wid * _CHUNK
    pltpu.sync_copy(x_hbm.at[pl.ds(base, _CHUNK)], idx_v)
    pltpu.sync_copy(w_hbm, w_v)
    lane3 = lax.iota(jnp.int32, 16) * _DIM

    def it(i, carry):
        idx = idx_v[pl.ds(i * 16, 16)]
        g = idx * _DIM
        p = lane3 + i * (16 * _DIM)
        for d in range(_DIM):
            vals = plsc.load_gather(w_v, [g + d])
            plsc.store_scatter(out_v, [p + d], vals)
        return carry

    lax.fori_loop(0, _NVEC, it, 0)
    pltpu.sync_copy(out_v, out_hbm.at[pl.ds(base * _DIM, _CHUNK * _DIM)])


def kernel(x, W):
    xf = x.reshape(-1).astype(jnp.int32)
    wf = jnp.pad(W.reshape(-1), (0, 32 - _TABLE_ROWS * _DIM))
    out = _sc_lookup(xf, wf)
    return out.reshape(_ROWS, _COLS, _DIM)


# speedup vs baseline: 42.5289x; 10.2878x over previous
import functools

import jax
import jax.numpy as jnp
from jax import lax
from jax.experimental import pallas as pl
from jax.experimental.pallas import tpu as pltpu
from jax.experimental.pallas import tpu_sc as plsc

_ROWS = 16384
_COLS = 26
_DIM = 3
_TABLE_ROWS = 10
_NW = 32
_CHUNK = _ROWS // _NW
_NVEC = _CHUNK // 16


@functools.partial(
    pl.kernel,
    out_type=jax.ShapeDtypeStruct((_DIM, _COLS, _ROWS), jnp.float32),
    mesh=plsc.VectorSubcoreMesh(core_axis_name="c", subcore_axis_name="s"),
    compiler_params=pltpu.CompilerParams(needs_layout_passes=False),
    scratch_types=[
        pltpu.VMEM((_COLS, _CHUNK), jnp.int32),
        pltpu.VMEM((_DIM, _COLS, _CHUNK), jnp.float32),
        pltpu.VMEM((_DIM * 16,), jnp.float32),
    ],
)
def _sc_lookup(x_hbm, w_hbm, out_hbm, idx_v, out_v, w_v):
    wid = lax.axis_index("s") * 2 + lax.axis_index("c")
    base = wid * _CHUNK
    pltpu.sync_copy(x_hbm.at[:, pl.ds(base, _CHUNK)], idx_v)
    pltpu.sync_copy(w_hbm, w_v)

    def col(j, carry):
        def vec(v, carry2):
            idx = idx_v[j, pl.ds(v * 16, 16)]
            for d in range(_DIM):
                out_v[d, j, pl.ds(v * 16, 16)] = plsc.load_gather(
                    w_v, [idx + (d * 16)]
                )
            return carry2

        return lax.fori_loop(0, _NVEC, vec, carry)

    lax.fori_loop(0, _COLS, col, 0)
    pltpu.sync_copy(out_v, out_hbm.at[:, :, pl.ds(base, _CHUNK)])


def kernel(x, W):
    xt = x.T.astype(jnp.int32)
    wt = jnp.pad(W.T, ((0, 0), (0, 16 - _TABLE_ROWS))).reshape(-1)
    out = _sc_lookup(xt, wt)
    return out.transpose(2, 1, 0)


# speedup vs baseline: 47.5677x; 1.1185x over previous
import functools

import jax
import jax.numpy as jnp
from jax import lax
from jax.experimental import pallas as pl
from jax.experimental.pallas import tpu as pltpu
from jax.experimental.pallas import tpu_sc as plsc

_ROWS = 16384
_COLS = 26
_DIM = 3
_TABLE_ROWS = 10
_NW = 32
_CHUNK = _ROWS // _NW
_NVEC = _CHUNK // 16


@functools.partial(
    pl.kernel,
    out_type=jax.ShapeDtypeStruct((_DIM, _COLS, _ROWS), jnp.float32),
    mesh=plsc.VectorSubcoreMesh(core_axis_name="c", subcore_axis_name="s"),
    compiler_params=pltpu.CompilerParams(needs_layout_passes=False),
    scratch_types=[
        pltpu.VMEM((_COLS, _CHUNK), jnp.int32),
        pltpu.VMEM((_DIM, _COLS, _CHUNK), jnp.float32),
        pltpu.VMEM((_DIM * 16,), jnp.float32),
    ],
)
def _sc_lookup(x_hbm, w_hbm, out_hbm, idx_v, out_v, w_v):
    wid = lax.axis_index("s") * 2 + lax.axis_index("c")
    base = wid * _CHUNK
    pltpu.sync_copy(x_hbm.at[:, pl.ds(base, _CHUNK)], idx_v)
    pltpu.sync_copy(w_hbm, w_v)

    def col(j, carry):
        for v in range(_NVEC):
            idx = idx_v[j, pl.ds(v * 16, 16)]
            for d in range(_DIM):
                out_v[d, j, pl.ds(v * 16, 16)] = plsc.load_gather(
                    w_v, [idx + (d * 16)]
                )
        return carry

    lax.fori_loop(0, _COLS, col, 0)
    pltpu.sync_copy(out_v, out_hbm.at[:, :, pl.ds(base, _CHUNK)])


def kernel(x, W):
    xt = x.T.astype(jnp.int32)
    wt = jnp.pad(W.T, ((0, 0), (0, 16 - _TABLE_ROWS))).reshape(-1)
    out = _sc_lookup(xt, wt)
    return out.transpose(2, 1, 0)


# speedup vs baseline: 61.1852x; 1.2863x over previous
import functools

import jax
import jax.numpy as jnp
from jax import lax
from jax.experimental import pallas as pl
from jax.experimental.pallas import tpu as pltpu
from jax.experimental.pallas import tpu_sc as plsc

_ROWS = 16384
_COLS = 26
_DIM = 3
_TABLE_ROWS = 10
_NW = 32
_CHUNK = _ROWS // _NW
_NVEC = _CHUNK // 16


def _vperm(table16, idx16):
    dnums = lax.GatherDimensionNumbers(
        offset_dims=(), collapsed_slice_dims=(0,), start_index_map=(0,)
    )
    return lax.gather(
        table16,
        idx16[:, None],
        dimension_numbers=dnums,
        slice_sizes=(1,),
        mode=lax.GatherScatterMode.PROMISE_IN_BOUNDS,
    )


@functools.partial(
    pl.kernel,
    out_type=jax.ShapeDtypeStruct((_DIM, _COLS, _ROWS), jnp.float32),
    mesh=plsc.VectorSubcoreMesh(core_axis_name="c", subcore_axis_name="s"),
    compiler_params=pltpu.CompilerParams(needs_layout_passes=False),
    scratch_types=[
        pltpu.VMEM((_COLS, _CHUNK), jnp.int32),
        pltpu.VMEM((_DIM, _COLS, _CHUNK), jnp.float32),
        pltpu.VMEM((_DIM * 16,), jnp.float32),
    ],
)
def _sc_lookup(x_hbm, w_hbm, out_hbm, idx_v, out_v, w_v):
    wid = lax.axis_index("s") * 2 + lax.axis_index("c")
    base = wid * _CHUNK
    pltpu.sync_copy(x_hbm.at[:, pl.ds(base, _CHUNK)], idx_v)
    pltpu.sync_copy(w_hbm, w_v)
    w_cols = [w_v[pl.ds(d * 16, 16)] for d in range(_DIM)]

    def col(j, carry):
        for v in range(_NVEC):
            idx = idx_v[j, pl.ds(v * 16, 16)]
            for d in range(_DIM):
                out_v[d, j, pl.ds(v * 16, 16)] = _vperm(w_cols[d], idx)
        return carry

    lax.fori_loop(0, _COLS, col, 0)
    pltpu.sync_copy(out_v, out_hbm.at[:, :, pl.ds(base, _CHUNK)])


def kernel(x, W):
    xt = x.T.astype(jnp.int32)
    wt = jnp.pad(W.T, ((0, 0), (0, 16 - _TABLE_ROWS))).reshape(-1)
    out = _sc_lookup(xt, wt)
    return out.transpose(2, 1, 0)
